# Initial kernel scaffold; baseline (speedup 1.0000x reference)
import jax
import jax.numpy as jnp
from jax.experimental import pallas as pl


def kernel(base_weights, x, target_mask_rate):
    raise NotImplementedError("write your pallas kernel here")


# speedup vs baseline: 99.3627x; 99.3627x over previous
import functools

import jax
import jax.numpy as jnp
from jax import lax
from jax.experimental import pallas as pl
from jax.experimental.pallas import tpu as pltpu
from jax.experimental.pallas import tpu_sc as plsc

_P = 0.02
_NC = 2
_NS = 16
_NW = _NC * _NS
_CHUNK = 2048


def _gather_sc(table, idx_flat):
    n = idx_flat.shape[0]
    per_w = n // _NW
    mesh = plsc.VectorSubcoreMesh(core_axis_name="c", subcore_axis_name="s")

    @functools.partial(
        pl.kernel,
        out_type=jax.ShapeDtypeStruct((n,), jnp.float32),
        mesh=mesh,
        scratch_types=[
            pltpu.VMEM((_CHUNK,), jnp.int32),
            pltpu.VMEM((_CHUNK,), jnp.float32),
            pltpu.SemaphoreType.DMA,
        ],
    )
    def gather_kernel(table_hbm, idx_hbm, out_hbm, idx_v, val_v, sem):
        wid = lax.axis_index("s") * _NC + lax.axis_index("c")
        base = wid * per_w

        @pl.loop(0, per_w, step=_CHUNK)
        def _(off):
            pltpu.sync_copy(idx_hbm.at[pl.ds(base + off, _CHUNK)], idx_v)
            pltpu.async_copy(table_hbm.at[idx_v], val_v, sem).wait()
            pltpu.sync_copy(val_v, out_hbm.at[pl.ds(base + off, _CHUNK)])

    return gather_kernel(table, idx_flat)


def _finish_tc(w, t):
    b, s = w.shape
    blk = 1024

    def body(w_ref, t_ref, o_ref):
        wv = w_ref[...]
        soft = jnp.exp(_P * jnp.log(wv))
        mu = jnp.mean(soft, axis=1, keepdims=True)
        tv = t_ref[...]
        down = soft * (tv / (mu + 1e-10))
        up = 1.0 - (1.0 - soft) * ((1.0 - tv) / (1.0 - mu + 1e-10))
        o_ref[...] = jnp.clip(jnp.where(mu > tv, down, up), 0.0, 1.0)

    return pl.pallas_call(
        body,
        grid=(b // blk,),
        in_specs=[
            pl.BlockSpec((blk, s), lambda i: (i, 0)),
            pl.BlockSpec((blk, 1), lambda i: (i, 0)),
        ],
        out_specs=pl.BlockSpec((blk, s), lambda i: (i, 0)),
        out_shape=jax.ShapeDtypeStruct((b, s), jnp.float32),
    )(w, t)


def kernel(base_weights, x, target_mask_rate):
    b, s = x.shape
    w_flat = _gather_sc(base_weights, x.reshape(-1))
    return _finish_tc(w_flat.reshape(b, s), target_mask_rate)
